# baseline (device time: 38451 ns/iter reference)
import jax
import jax.numpy as jnp
from jax import lax
from jax.experimental import pallas as pl
from jax.experimental.pallas import tpu as pltpu

B, H, D, BS = 16, 16, 64, 16
NB = 128
NP = 128
NSUB = 8
PAGES_SUB = NP // NSUB
KEYS_SUB = PAGES_SUB * BS
SCALE = D ** -0.5
HD = H * D


def kernel(Q, K, V, bt, lens):
    lens2 = lens.reshape(B, 1)
    sub_idx = lax.axis_index("y") * 4 + lax.axis_index("z")
    Ksub = lax.dynamic_slice_in_dim(
        K, sub_idx * PAGES_SUB, PAGES_SUB, 0).astype(jnp.bfloat16)
    Vsub = lax.dynamic_slice_in_dim(
        V, sub_idx * PAGES_SUB, PAGES_SUB, 0).astype(jnp.bfloat16)

    def body(q_ref, k_ref, v_ref, bt_ref, lens_ref, out_ref,
             comm_acc, comm_ml, send_sems, recv_sems):
        my_x = lax.axis_index("x")
        my_y = lax.axis_index("y")
        my_z = lax.axis_index("z")
        sub = my_y * 4 + my_z

        page0 = my_x * NP + sub * PAGES_SUB
        bt3 = bt_ref[:][:, :, None]
        gid3 = lax.broadcasted_iota(jnp.int32, (B, NB, PAGES_SUB), 2) + page0
        pos3 = lax.broadcasted_iota(jnp.int32, (B, NB, PAGES_SUB), 1)
        valid = (bt3 == gid3) & (pos3 < lens_ref[:][:, :, None])
        cnt = jnp.sum(valid.astype(jnp.float32), axis=1)

        kk = lax.broadcasted_iota(jnp.int32, (PAGES_SUB, KEYS_SUB), 1) // BS
        pp = lax.broadcasted_iota(jnp.int32, (PAGES_SUB, KEYS_SUB), 0)
        R = (kk == pp).astype(jnp.float32)
        w = lax.dot_general(cnt, R, (((1,), (0,)), ((), ())),
                            preferred_element_type=jnp.float32)

        hh = lax.broadcasted_iota(jnp.int32, (H, HD), 1) // D
        h0 = lax.broadcasted_iota(jnp.int32, (H, HD), 0)
        RH = (hh == h0).astype(jnp.float32)

        kb = k_ref[:].reshape(KEYS_SUB, HD)
        vb = v_ref[:].reshape(KEYS_SUB, HD)

        acc_list, m_list, l_list = [], [], []
        for h in range(H):
            sl = slice(h * D, (h + 1) * D)
            qh = (q_ref[:, 0, h, :] * SCALE).astype(jnp.bfloat16)
            s = lax.dot_general(qh, kb[:, sl],
                                (((1,), (1,)), ((), ())),
                                preferred_element_type=jnp.float32)
            m_h = jnp.max(s, axis=1, keepdims=True)
            p = jnp.exp(s - m_h) * w
            l_h = jnp.sum(p, axis=1, keepdims=True)
            acc_h = lax.dot_general(p.astype(jnp.bfloat16), vb[:, sl],
                                    (((1,), (0,)), ((), ())),
                                    preferred_element_type=jnp.float32)
            acc_list.append(acc_h)
            m_list.append(m_h)
            l_list.append(l_h)
        cur_acc = jnp.concatenate(acc_list, axis=1)
        cur_m = jnp.concatenate(m_list, axis=1)
        cur_l = jnp.concatenate(l_list, axis=1)

        zpartners = [(my_x, my_y, my_z ^ m) for m in (1, 2, 3)]
        ypartner = (my_x, 1 - my_y, my_z)
        xpartner = (1 - my_x, my_y, my_z)

        barrier_sem = pltpu.get_barrier_semaphore()
        for prt in zpartners + [ypartner, xpartner]:
            pl.semaphore_signal(barrier_sem, inc=1, device_id=prt,
                                device_id_type=pl.DeviceIdType.MESH)
        pl.semaphore_wait(barrier_sem, 5)

        pending = []

        def expand(f):
            return lax.dot_general(f, RH, (((1,), (0,)), ((), ())),
                                   preferred_element_type=jnp.float32)

        comm_acc[0] = cur_acc
        comm_ml[0, 0] = cur_m
        comm_ml[0, 1] = cur_l
        zrdmas = []
        for j, prt in enumerate(zpartners):
            r_acc = pltpu.make_async_remote_copy(
                src_ref=comm_acc.at[0], dst_ref=comm_acc.at[1 + j],
                send_sem=send_sems.at[j], recv_sem=recv_sems.at[j],
                device_id=prt, device_id_type=pl.DeviceIdType.MESH)
            r_ml = pltpu.make_async_remote_copy(
                src_ref=comm_ml.at[0], dst_ref=comm_ml.at[1 + j],
                send_sem=send_sems.at[3 + j], recv_sem=recv_sems.at[3 + j],
                device_id=prt, device_id_type=pl.DeviceIdType.MESH)
            r_acc.start()
            r_ml.start()
            zrdmas.append((r_acc, r_ml))
        for r_acc, r_ml in zrdmas:
            r_acc.wait_recv()
            r_ml.wait_recv()
            pending.append((r_acc, r_ml))
        ms = [cur_m] + [comm_ml[1 + j, 0] for j in range(3)]
        ls = [cur_l] + [comm_ml[1 + j, 1] for j in range(3)]
        accs = [cur_acc] + [comm_acc[1 + j] for j in range(3)]
        m_new = jnp.maximum(jnp.maximum(ms[0], ms[1]),
                            jnp.maximum(ms[2], ms[3]))
        fs = [jnp.exp(m - m_new) for m in ms]
        cur_l = sum(l * f for l, f in zip(ls, fs))
        cur_acc = sum(a * expand(f) for a, f in zip(accs, fs))
        cur_m = m_new

        for s_i, prt in ((0, ypartner), (1, xpartner)):
            snd, rcv = 4 + 2 * s_i, 5 + 2 * s_i
            sem_a, sem_m = 6 + 2 * s_i, 7 + 2 * s_i
            comm_acc[snd] = cur_acc
            comm_ml[snd, 0] = cur_m
            comm_ml[snd, 1] = cur_l
            r_acc = pltpu.make_async_remote_copy(
                src_ref=comm_acc.at[snd], dst_ref=comm_acc.at[rcv],
                send_sem=send_sems.at[sem_a], recv_sem=recv_sems.at[sem_a],
                device_id=prt, device_id_type=pl.DeviceIdType.MESH)
            r_ml = pltpu.make_async_remote_copy(
                src_ref=comm_ml.at[snd], dst_ref=comm_ml.at[rcv],
                send_sem=send_sems.at[sem_m], recv_sem=recv_sems.at[sem_m],
                device_id=prt, device_id_type=pl.DeviceIdType.MESH)
            r_acc.start()
            r_ml.start()
            r_acc.wait_recv()
            r_ml.wait_recv()
            pending.append((r_acc, r_ml))

            m_rmt = comm_ml[rcv, 0]
            l_rmt = comm_ml[rcv, 1]
            acc_rmt = comm_acc[rcv]
            m_new = jnp.maximum(cur_m, m_rmt)
            fa = jnp.exp(cur_m - m_new)
            fb = jnp.exp(m_rmt - m_new)
            cur_l = cur_l * fa + l_rmt * fb
            cur_acc = cur_acc * expand(fa) + acc_rmt * expand(fb)
            cur_m = m_new

        inv_l = 1.0 / cur_l
        for h in range(H):
            out_ref[:, 0, h, :] = (cur_acc[:, h * D:(h + 1) * D]
                                   * inv_l[:, h:h + 1])

        for r_acc, r_ml in pending:
            r_acc.wait_send()
            r_ml.wait_send()

    return pl.pallas_call(
        body,
        out_shape=jax.ShapeDtypeStruct((B, 1, H, D), jnp.float32),
        in_specs=[pl.BlockSpec(memory_space=pltpu.MemorySpace.VMEM)] * 5,
        out_specs=pl.BlockSpec(memory_space=pltpu.MemorySpace.VMEM),
        scratch_shapes=[
            pltpu.VMEM((8, B, HD), jnp.float32),
            pltpu.VMEM((8, 2, B, H), jnp.float32),
            pltpu.SemaphoreType.DMA((10,)),
            pltpu.SemaphoreType.DMA((10,)),
        ],
        compiler_params=pltpu.CompilerParams(
            collective_id=0, vmem_limit_bytes=100 * 1024 * 1024),
    )(Q, Ksub, Vsub, bt, lens2)
